# Initial kernel scaffold; baseline (speedup 1.0000x reference)
import jax
import jax.numpy as jnp
from jax.experimental import pallas as pl


def kernel(x, adj, W1p0, W1p1, W1e0, W1e1, W2p0, W2p1, W2e0, W2e1, W3a, W3b):
    raise NotImplementedError("write your pallas kernel here")


# speedup vs baseline: 1.1338x; 1.1338x over previous
import jax
import jax.numpy as jnp
from jax.experimental import pallas as pl

B = 8
MAXN = 1024
HID = 64
OUT = 16
N1 = 103
N2 = 11


def _mm(a, b):
    return jax.lax.dot_general(a, b, (((1,), (0,)), ((), ())),
                               preferred_element_type=jnp.float32)


def _mm_t(a, b):
    return jax.lax.dot_general(a, b, (((0,), (0,)), ((), ())),
                               preferred_element_type=jnp.float32)


def _softmax(z):
    z = z - jnp.max(z, axis=-1, keepdims=True)
    e = jnp.exp(z)
    return e / jnp.sum(e, axis=-1, keepdims=True)


def _diffpool_body(x_ref, adj_ref, W1p0_ref, W1p1_ref, W1e0_ref, W1e1_ref,
                   W2p0_ref, W2p1_ref, W2e0_ref, W2e1_ref, W3a_ref, W3b_ref,
                   out_ref):
    x = x_ref[0]
    adj = adj_ref[0]

    t = _mm(adj, x)
    s1 = jax.nn.relu(_mm(t, W1p0_ref[...]))
    h1 = jax.nn.relu(_mm(t, W1e0_ref[...]))
    s = jax.nn.relu(_mm(_mm(adj, s1), W1p1_ref[...]))
    h = jax.nn.relu(_mm(_mm(adj, h1), W1e1_ref[...]))

    ss = _softmax(s)
    x_p = _mm_t(ss, h)
    a_p = _mm_t(ss, _mm(adj, ss))

    t2 = _mm(a_p, x_p)
    s2a = jax.nn.relu(_mm(t2, W2p0_ref[...]))
    h2a = jax.nn.relu(_mm(t2, W2e0_ref[...]))
    s2 = jax.nn.relu(_mm(_mm(a_p, s2a), W2p1_ref[...]))
    h2 = jax.nn.relu(_mm(_mm(a_p, h2a), W2e1_ref[...]))

    ss2 = _softmax(s2)
    x_q = _mm_t(ss2, h2)
    a_q = _mm_t(ss2, _mm(a_p, ss2))

    g = jax.nn.relu(_mm(_mm(a_q, x_q), W3a_ref[...]))
    g = jax.nn.relu(_mm(_mm(a_q, g), W3b_ref[...]))
    out_ref[pl.program_id(0)] = jnp.mean(g, axis=0)


def kernel(x, adj, W1p0, W1p1, W1e0, W1e1, W2p0, W2p1, W2e0, W2e1, W3a, W3b):
    w_spec = lambda shp: pl.BlockSpec(shp, lambda b: (0,) * len(shp))
    return pl.pallas_call(
        _diffpool_body,
        grid=(B,),
        in_specs=[
            pl.BlockSpec((1, MAXN, HID), lambda b: (b, 0, 0)),
            pl.BlockSpec((1, MAXN, MAXN), lambda b: (b, 0, 0)),
            w_spec(W1p0.shape), w_spec(W1p1.shape),
            w_spec(W1e0.shape), w_spec(W1e1.shape),
            w_spec(W2p0.shape), w_spec(W2p1.shape),
            w_spec(W2e0.shape), w_spec(W2e1.shape),
            w_spec(W3a.shape), w_spec(W3b.shape),
        ],
        out_specs=pl.BlockSpec((B, OUT), lambda b: (0, 0)),
        out_shape=jax.ShapeDtypeStruct((B, OUT), jnp.float32),
    )(x, adj, W1p0, W1p1, W1e0, W1e1, W2p0, W2p1, W2e0, W2e1, W3a, W3b)
